# Initial kernel scaffold; baseline (speedup 1.0000x reference)
import jax
import jax.numpy as jnp
from jax.experimental import pallas as pl


def kernel(x, edge_index, W1, b1, g1, be1, W2, b2, g2, be2, Wc, bc):
    raise NotImplementedError("write your pallas kernel here")


# speedup vs baseline: 21.8465x; 21.8465x over previous
import functools

import jax
import jax.numpy as jnp
from jax import lax
from jax.experimental import pallas as pl
from jax.experimental.pallas import tpu as pltpu
from jax.experimental.pallas import tpu_sc as plsc

N = 10000
D = 128
H = 128
O = 40
NC = 2
NS = 16
NW = NC * NS
K = 128
ROWS_PER_TILE = 632
ACC_ROWS = NS * ROWS_PER_TILE
PAD_ROWS = ACC_ROWS - N
BB = 1000


def _sc_mesh():
    return plsc.VectorSubcoreMesh(core_axis_name="c", subcore_axis_name="s")


def _deg_body(nchunk, dst_hbm, out0, out1, dstb, onesb, acc, zbuf):
    cid = lax.axis_index("c")
    sid = lax.axis_index("s")
    wid = sid * NC + cid

    def fill_ones(j, _):
        onesb[j] = jnp.full((16,), 1.0, jnp.float32)
        return 0

    lax.fori_loop(0, K, fill_ones, 0)

    def fill_zeros(j, _):
        zbuf[j] = jnp.zeros((16,), jnp.float32)
        return 0

    lax.fori_loop(0, ROWS_PER_TILE, fill_zeros, 0)
    sl = pl.ds(sid * ROWS_PER_TILE, ROWS_PER_TILE)
    pltpu.sync_copy(zbuf, acc.at[sl])
    plsc.subcore_barrier()

    pltpu.sync_copy(dst_hbm.at[wid], dstb)

    def body(j, _):
        pltpu.sync_copy(onesb, acc.at[dstb.at[j]], add=True)
        return 0

    lax.fori_loop(0, nchunk, body, 0)
    plsc.subcore_barrier()

    @pl.when(cid == 0)
    def _():
        pltpu.sync_copy(acc.at[sl], out0.at[sl])

    @pl.when(cid == 1)
    def _():
        pltpu.sync_copy(acc.at[sl], out1.at[sl])


def _make_deg_kernel(nchunk):
    return functools.partial(
        pl.kernel,
        out_type=(
            jax.ShapeDtypeStruct((ACC_ROWS, 16), jnp.float32),
            jax.ShapeDtypeStruct((ACC_ROWS, 16), jnp.float32),
        ),
        mesh=_sc_mesh(),
        scratch_types=[
            pltpu.VMEM((nchunk, K), jnp.int32),
            pltpu.VMEM((K, 16), jnp.float32),
            pltpu.VMEM_SHARED((ACC_ROWS, 16), jnp.float32),
            pltpu.VMEM((ROWS_PER_TILE, 16), jnp.float32),
        ],
    )(functools.partial(_deg_body, nchunk))


def _mp_body(nchunk, hp_hbm, src_hbm, dst_hbm, out0, out1,
             srcb, dstb, rows, acc, gsem):
    cid = lax.axis_index("c")
    sid = lax.axis_index("s")
    wid = sid * NC + cid
    sl = pl.ds(sid * ROWS_PER_TILE, ROWS_PER_TILE)

    pltpu.sync_copy(hp_hbm.at[sl], acc.at[sl])
    plsc.subcore_barrier()

    pltpu.sync_copy(src_hbm.at[wid], srcb)
    pltpu.sync_copy(dst_hbm.at[wid], dstb)

    def body(j, _):
        pltpu.async_copy(hp_hbm.at[srcb.at[j]], rows, gsem).wait()
        pltpu.sync_copy(rows, acc.at[dstb.at[j]], add=True)
        return 0

    lax.fori_loop(0, nchunk, body, 0)
    plsc.subcore_barrier()

    @pl.when(cid == 0)
    def _():
        pltpu.sync_copy(acc.at[sl], out0.at[sl])

    @pl.when(cid == 1)
    def _():
        pltpu.sync_copy(acc.at[sl], out1.at[sl])


def _make_mp_kernel(nchunk):
    return functools.partial(
        pl.kernel,
        out_type=(
            jax.ShapeDtypeStruct((ACC_ROWS, D), jnp.float32),
            jax.ShapeDtypeStruct((ACC_ROWS, D), jnp.float32),
        ),
        mesh=_sc_mesh(),
        scratch_types=[
            pltpu.VMEM((nchunk, K), jnp.int32),
            pltpu.VMEM((nchunk, K), jnp.int32),
            pltpu.VMEM((K, D), jnp.float32),
            pltpu.VMEM_SHARED((ACC_ROWS, D), jnp.float32),
            pltpu.SemaphoreType.DMA,
        ],
    )(functools.partial(_mp_body, nchunk))


def _dense1_body(x_ref, c0_ref, c1_ref, w_ref, hp_ref, dinv_ref):
    cnt = c0_ref[:, 0:1] + c1_ref[:, 0:1]
    dinv = lax.rsqrt(cnt + 2.0)
    h = jnp.dot(x_ref[...], w_ref[...], preferred_element_type=jnp.float32)
    hp_ref[...] = dinv * h
    dinv_ref[...] = dinv


def _post1_body(s0_ref, s1_ref, dinv_ref, b_ref, g_ref, be_ref, w_ref,
                x1_ref, hp2_ref):
    dinv = dinv_ref[...]
    t = dinv * (s0_ref[...] + s1_ref[...]) + b_ref[...]
    mu = jnp.mean(t, axis=1, keepdims=True)
    var = jnp.mean((t - mu) ** 2, axis=1, keepdims=True)
    tn = (t - mu) * lax.rsqrt(var + 1e-5) * g_ref[...] + be_ref[...]
    x1 = jnp.maximum(tn, 0.0)
    x1_ref[...] = x1
    h2 = jnp.dot(x1, w_ref[...], preferred_element_type=jnp.float32)
    hp2_ref[...] = dinv * h2


def _post2_body(s0_ref, s1_ref, dinv_ref, b_ref, g_ref, be_ref, x1_ref,
                wc_ref, bc_ref, out_ref):
    t = dinv_ref[...] * (s0_ref[...] + s1_ref[...]) + b_ref[...]
    mu = jnp.mean(t, axis=1, keepdims=True)
    var = jnp.mean((t - mu) ** 2, axis=1, keepdims=True)
    tn = (t - mu) * lax.rsqrt(var + 1e-5) * g_ref[...] + be_ref[...]
    h = jnp.maximum(tn, 0.0) + 0.2 * x1_ref[...]
    logits = jnp.dot(h, wc_ref[...], preferred_element_type=jnp.float32)
    logits = logits + bc_ref[...]
    m = jnp.max(logits, axis=1, keepdims=True)
    lse = jnp.log(jnp.sum(jnp.exp(logits - m), axis=1, keepdims=True)) + m
    out_ref[...] = logits - lse


def _row_spec(bs):
    return pl.BlockSpec(bs, lambda i: (i, 0))


def _full_spec(bs):
    return pl.BlockSpec(bs, lambda i: (0, 0))


def kernel(x, edge_index, W1, b1, g1, be1, W2, b2, g2, be2, Wc, bc):
    E = edge_index.shape[1]
    epad = ((E + NW * K - 1) // (NW * K)) * (NW * K)
    pad_n = epad - E
    nchunk = epad // (NW * K)

    ar = jnp.arange(pad_n, dtype=jnp.int32)
    src_pad = ar % jnp.int32(N)
    dst_pad = jnp.int32(N) + (ar % jnp.int32(PAD_ROWS))
    src = jnp.concatenate([edge_index[0], src_pad]).reshape(NW, nchunk, K)
    dst = jnp.concatenate([edge_index[1], dst_pad]).reshape(NW, nchunk, K)

    c0, c1 = _make_deg_kernel(nchunk)(dst)

    grid = N // BB
    W1t = W1.T
    W2t = W2.T
    Wct = Wc.T
    b1r, g1r, be1r = b1.reshape(1, H), g1.reshape(1, H), be1.reshape(1, H)
    b2r, g2r, be2r = b2.reshape(1, H), g2.reshape(1, H), be2.reshape(1, H)
    bcr = bc.reshape(1, O)

    hp1, dinv = pl.pallas_call(
        _dense1_body,
        grid=(grid,),
        in_specs=[_row_spec((BB, D)), _row_spec((BB, 16)), _row_spec((BB, 16)),
                  _full_spec((D, H))],
        out_specs=[_row_spec((BB, H)), _row_spec((BB, 1))],
        out_shape=[jax.ShapeDtypeStruct((ACC_ROWS, H), jnp.float32),
                   jax.ShapeDtypeStruct((N, 1), jnp.float32)],
    )(x, c0, c1, W1t)

    s0, s1 = _make_mp_kernel(nchunk)(hp1, src, dst)

    x1, hp2 = pl.pallas_call(
        _post1_body,
        grid=(grid,),
        in_specs=[_row_spec((BB, H)), _row_spec((BB, H)), _row_spec((BB, 1)),
                  _full_spec((1, H)), _full_spec((1, H)), _full_spec((1, H)),
                  _full_spec((H, H))],
        out_specs=[_row_spec((BB, H)), _row_spec((BB, H))],
        out_shape=[jax.ShapeDtypeStruct((N, H), jnp.float32),
                   jax.ShapeDtypeStruct((ACC_ROWS, H), jnp.float32)],
    )(s0, s1, dinv, b1r, g1r, be1r, W2t)

    t0, t1 = _make_mp_kernel(nchunk)(hp2, src, dst)

    out = pl.pallas_call(
        _post2_body,
        grid=(grid,),
        in_specs=[_row_spec((BB, H)), _row_spec((BB, H)), _row_spec((BB, 1)),
                  _full_spec((1, H)), _full_spec((1, H)), _full_spec((1, H)),
                  _row_spec((BB, H)), _full_spec((H, O)), _full_spec((1, O))],
        out_specs=_row_spec((BB, O)),
        out_shape=jax.ShapeDtypeStruct((N, O), jnp.float32),
    )(t0, t1, dinv, b2r, g2r, be2r, x1, Wct, bcr)

    return out


# speedup vs baseline: 25.4711x; 1.1659x over previous
import functools

import jax
import jax.numpy as jnp
from jax import lax
from jax.experimental import pallas as pl
from jax.experimental.pallas import tpu as pltpu
from jax.experimental.pallas import tpu_sc as plsc

N = 10000
D = 128
H = 128
O = 40
NC = 2
NS = 16
NW = NC * NS
K = 96
ROWS_PER_TILE = 632
ACC_ROWS = NS * ROWS_PER_TILE
PAD_ROWS = ACC_ROWS - N
BB = 1000


def _sc_mesh():
    return plsc.VectorSubcoreMesh(core_axis_name="c", subcore_axis_name="s")


def _deg_body(nchunk, dst_hbm, out0, out1, dstb, onesb, acc, zbuf, dsem):
    cid = lax.axis_index("c")
    sid = lax.axis_index("s")
    wid = sid * NC + cid

    def fill_ones(j, _):
        onesb[j] = jnp.full((16,), 1.0, jnp.float32)
        return 0

    lax.fori_loop(0, K, fill_ones, 0)

    def fill_zeros(j, _):
        zbuf[j] = jnp.zeros((16,), jnp.float32)
        return 0

    lax.fori_loop(0, ROWS_PER_TILE, fill_zeros, 0)
    sl = pl.ds(sid * ROWS_PER_TILE, ROWS_PER_TILE)
    pltpu.sync_copy(zbuf, acc.at[sl])
    plsc.subcore_barrier()

    pltpu.sync_copy(dst_hbm.at[wid], dstb)

    def body(j, _):
        pltpu.async_copy(onesb, acc.at[dstb.at[j]], dsem, add=True)

        @pl.when(j > 0)
        def _():
            pltpu.make_async_copy(onesb, acc.at[dstb.at[0]], dsem).wait()

        return 0

    lax.fori_loop(0, nchunk, body, 0)
    pltpu.make_async_copy(onesb, acc.at[dstb.at[0]], dsem).wait()
    plsc.subcore_barrier()

    @pl.when(cid == 0)
    def _():
        pltpu.sync_copy(acc.at[sl], out0.at[sl])

    @pl.when(cid == 1)
    def _():
        pltpu.sync_copy(acc.at[sl], out1.at[sl])


def _make_deg_kernel(nchunk):
    return functools.partial(
        pl.kernel,
        out_type=(
            jax.ShapeDtypeStruct((ACC_ROWS, 16), jnp.float32),
            jax.ShapeDtypeStruct((ACC_ROWS, 16), jnp.float32),
        ),
        mesh=_sc_mesh(),
        scratch_types=[
            pltpu.VMEM((nchunk, K), jnp.int32),
            pltpu.VMEM((K, 16), jnp.float32),
            pltpu.VMEM_SHARED((ACC_ROWS, 16), jnp.float32),
            pltpu.VMEM((ROWS_PER_TILE, 16), jnp.float32),
            pltpu.SemaphoreType.DMA,
        ],
    )(functools.partial(_deg_body, nchunk))


def _mp_body(nchunk, hp_hbm, src_hbm, dst_hbm, out0, out1,
             srcb, dstb, rows0, rows1, acc, g0, g1):
    cid = lax.axis_index("c")
    sid = lax.axis_index("s")
    wid = sid * NC + cid
    sl = pl.ds(sid * ROWS_PER_TILE, ROWS_PER_TILE)

    pltpu.sync_copy(hp_hbm.at[sl], acc.at[sl])
    plsc.subcore_barrier()

    pltpu.sync_copy(src_hbm.at[wid], srcb)
    pltpu.sync_copy(dst_hbm.at[wid], dstb)

    def gather(j, rows, sem):
        pltpu.async_copy(hp_hbm.at[srcb.at[pl.ds(j * K, K)]], rows, sem)

    def wait_gather(rows, sem):
        pltpu.make_async_copy(hp_hbm.at[srcb.at[pl.ds(0, K)]], rows, sem).wait()

    def scatter(j, rows):
        pltpu.sync_copy(rows, acc.at[dstb.at[j]], add=True)

    npairs = (nchunk - 1) // 2
    gather(0, rows0, g0)

    def pair(t, _):
        j = 2 * t
        wait_gather(rows0, g0)
        gather(j + 1, rows1, g1)
        scatter(j, rows0)
        wait_gather(rows1, g1)
        gather(j + 2, rows0, g0)
        scatter(j + 1, rows1)
        return 0

    lax.fori_loop(0, npairs, pair, 0)
    wait_gather(rows0, g0)
    scatter(nchunk - 1, rows0)
    plsc.subcore_barrier()

    @pl.when(cid == 0)
    def _():
        pltpu.sync_copy(acc.at[sl], out0.at[sl])

    @pl.when(cid == 1)
    def _():
        pltpu.sync_copy(acc.at[sl], out1.at[sl])


def _make_mp_kernel(nchunk):
    return functools.partial(
        pl.kernel,
        out_type=(
            jax.ShapeDtypeStruct((ACC_ROWS, D), jnp.float32),
            jax.ShapeDtypeStruct((ACC_ROWS, D), jnp.float32),
        ),
        mesh=_sc_mesh(),
        scratch_types=[
            pltpu.VMEM((nchunk * K,), jnp.int32),
            pltpu.VMEM((nchunk, K), jnp.int32),
            pltpu.VMEM((K, D), jnp.float32),
            pltpu.VMEM((K, D), jnp.float32),
            pltpu.VMEM_SHARED((ACC_ROWS, D), jnp.float32),
            pltpu.SemaphoreType.DMA,
            pltpu.SemaphoreType.DMA,
        ],
    )(functools.partial(_mp_body, nchunk))


def _dense1_body(x_ref, c0_ref, c1_ref, w_ref, hp_ref, dinv_ref):
    cnt = c0_ref[:, 0:1] + c1_ref[:, 0:1]
    dinv = lax.rsqrt(cnt + 2.0)
    h = jnp.dot(x_ref[...], w_ref[...], preferred_element_type=jnp.float32)
    hp_ref[...] = dinv * h
    dinv_ref[...] = dinv


def _post1_body(s0_ref, s1_ref, dinv_ref, b_ref, g_ref, be_ref, w_ref,
                x1_ref, hp2_ref):
    dinv = dinv_ref[...]
    t = dinv * (s0_ref[...] + s1_ref[...]) + b_ref[...]
    mu = jnp.mean(t, axis=1, keepdims=True)
    var = jnp.mean((t - mu) ** 2, axis=1, keepdims=True)
    tn = (t - mu) * lax.rsqrt(var + 1e-5) * g_ref[...] + be_ref[...]
    x1 = jnp.maximum(tn, 0.0)
    x1_ref[...] = x1
    h2 = jnp.dot(x1, w_ref[...], preferred_element_type=jnp.float32)
    hp2_ref[...] = dinv * h2


def _post2_body(s0_ref, s1_ref, dinv_ref, b_ref, g_ref, be_ref, x1_ref,
                wc_ref, bc_ref, out_ref):
    t = dinv_ref[...] * (s0_ref[...] + s1_ref[...]) + b_ref[...]
    mu = jnp.mean(t, axis=1, keepdims=True)
    var = jnp.mean((t - mu) ** 2, axis=1, keepdims=True)
    tn = (t - mu) * lax.rsqrt(var + 1e-5) * g_ref[...] + be_ref[...]
    h = jnp.maximum(tn, 0.0) + 0.2 * x1_ref[...]
    logits = jnp.dot(h, wc_ref[...], preferred_element_type=jnp.float32)
    logits = logits + bc_ref[...]
    m = jnp.max(logits, axis=1, keepdims=True)
    lse = jnp.log(jnp.sum(jnp.exp(logits - m), axis=1, keepdims=True)) + m
    out_ref[...] = logits - lse


def _row_spec(bs):
    return pl.BlockSpec(bs, lambda i: (i, 0))


def _full_spec(bs):
    return pl.BlockSpec(bs, lambda i: (0, 0))


def kernel(x, edge_index, W1, b1, g1, be1, W2, b2, g2, be2, Wc, bc):
    E = edge_index.shape[1]
    epad = ((E + NW * K - 1) // (NW * K)) * (NW * K)
    pad_n = epad - E
    nchunk = epad // (NW * K)
    if nchunk % 2 == 0:
        epad += NW * K
        pad_n += NW * K
        nchunk += 1

    ar = jnp.arange(pad_n, dtype=jnp.int32)
    src_pad = ar % jnp.int32(N)
    dst_pad = jnp.int32(N) + (ar % jnp.int32(PAD_ROWS))
    src = jnp.concatenate([edge_index[0], src_pad]).reshape(NW, nchunk * K)
    dst = jnp.concatenate([edge_index[1], dst_pad]).reshape(NW, nchunk, K)

    c0, c1 = _make_deg_kernel(nchunk)(dst)

    grid = N // BB
    W1t = W1.T
    W2t = W2.T
    Wct = Wc.T
    b1r, g1r, be1r = b1.reshape(1, H), g1.reshape(1, H), be1.reshape(1, H)
    b2r, g2r, be2r = b2.reshape(1, H), g2.reshape(1, H), be2.reshape(1, H)
    bcr = bc.reshape(1, O)

    hp1, dinv = pl.pallas_call(
        _dense1_body,
        grid=(grid,),
        in_specs=[_row_spec((BB, D)), _row_spec((BB, 16)), _row_spec((BB, 16)),
                  _full_spec((D, H))],
        out_specs=[_row_spec((BB, H)), _row_spec((BB, 1))],
        out_shape=[jax.ShapeDtypeStruct((ACC_ROWS, H), jnp.float32),
                   jax.ShapeDtypeStruct((N, 1), jnp.float32)],
    )(x, c0, c1, W1t)

    s0, s1 = _make_mp_kernel(nchunk)(hp1, src, dst)

    x1, hp2 = pl.pallas_call(
        _post1_body,
        grid=(grid,),
        in_specs=[_row_spec((BB, H)), _row_spec((BB, H)), _row_spec((BB, 1)),
                  _full_spec((1, H)), _full_spec((1, H)), _full_spec((1, H)),
                  _full_spec((H, H))],
        out_specs=[_row_spec((BB, H)), _row_spec((BB, H))],
        out_shape=[jax.ShapeDtypeStruct((N, H), jnp.float32),
                   jax.ShapeDtypeStruct((ACC_ROWS, H), jnp.float32)],
    )(s0, s1, dinv, b1r, g1r, be1r, W2t)

    t0, t1 = _make_mp_kernel(nchunk)(hp2, src, dst)

    out = pl.pallas_call(
        _post2_body,
        grid=(grid,),
        in_specs=[_row_spec((BB, H)), _row_spec((BB, H)), _row_spec((BB, 1)),
                  _full_spec((1, H)), _full_spec((1, H)), _full_spec((1, H)),
                  _row_spec((BB, H)), _full_spec((H, O)), _full_spec((1, O))],
        out_specs=_row_spec((BB, O)),
        out_shape=jax.ShapeDtypeStruct((N, O), jnp.float32),
    )(t0, t1, dinv, b2r, g2r, be2r, x1, Wct, bcr)

    return out


# speedup vs baseline: 25.4837x; 1.0005x over previous
import functools

import jax
import jax.numpy as jnp
from jax import lax
from jax.experimental import pallas as pl
from jax.experimental.pallas import tpu as pltpu
from jax.experimental.pallas import tpu_sc as plsc

N = 10000
D = 128
H = 128
O = 40
NC = 2
NS = 16
NW = NC * NS
K = 96
ROWS_PER_TILE = 632
ACC_ROWS = NS * ROWS_PER_TILE
PAD_ROWS = ACC_ROWS - N
BB = 1000


def _sc_mesh():
    return plsc.VectorSubcoreMesh(core_axis_name="c", subcore_axis_name="s")


def _deg_body(nchunk, dst_hbm, out0, out1, dstb, onesb, acc, zbuf, dsem):
    cid = lax.axis_index("c")
    sid = lax.axis_index("s")
    wid = sid * NC + cid

    def fill_ones(j, _):
        onesb[j] = jnp.full((16,), 1.0, jnp.float32)
        return 0

    lax.fori_loop(0, K, fill_ones, 0)

    def fill_zeros(j, _):
        zbuf[j] = jnp.zeros((16,), jnp.float32)
        return 0

    lax.fori_loop(0, ROWS_PER_TILE, fill_zeros, 0)
    sl = pl.ds(sid * ROWS_PER_TILE, ROWS_PER_TILE)
    pltpu.sync_copy(zbuf, acc.at[sl])
    plsc.subcore_barrier()

    pltpu.sync_copy(dst_hbm.at[wid], dstb)

    def body(j, _):
        pltpu.async_copy(onesb, acc.at[dstb.at[j]], dsem, add=True)

        @pl.when(j > 0)
        def _():
            pltpu.make_async_copy(onesb, acc.at[dstb.at[0]], dsem).wait()

        return 0

    lax.fori_loop(0, nchunk, body, 0)
    pltpu.make_async_copy(onesb, acc.at[dstb.at[0]], dsem).wait()
    plsc.subcore_barrier()

    @pl.when(cid == 0)
    def _():
        pltpu.sync_copy(acc.at[sl], out0.at[sl])

    @pl.when(cid == 1)
    def _():
        pltpu.sync_copy(acc.at[sl], out1.at[sl])


def _make_deg_kernel(nchunk):
    return functools.partial(
        pl.kernel,
        out_type=(
            jax.ShapeDtypeStruct((ACC_ROWS, 16), jnp.float32),
            jax.ShapeDtypeStruct((ACC_ROWS, 16), jnp.float32),
        ),
        mesh=_sc_mesh(),
        scratch_types=[
            pltpu.VMEM((nchunk, K), jnp.int32),
            pltpu.VMEM((K, 16), jnp.float32),
            pltpu.VMEM_SHARED((ACC_ROWS, 16), jnp.float32),
            pltpu.VMEM((ROWS_PER_TILE, 16), jnp.float32),
            pltpu.SemaphoreType.DMA,
        ],
    )(functools.partial(_deg_body, nchunk))


def _mp_body(nchunk, hp_hbm, src_hbm, dst_hbm, out0, out1,
             srcb, dstb, rows0, rows1, acc, g0, g1, s0, s1):
    cid = lax.axis_index("c")
    sid = lax.axis_index("s")
    wid = sid * NC + cid
    sl = pl.ds(sid * ROWS_PER_TILE, ROWS_PER_TILE)

    pltpu.sync_copy(hp_hbm.at[sl], acc.at[sl])
    plsc.subcore_barrier()

    pltpu.sync_copy(src_hbm.at[wid], srcb)
    pltpu.sync_copy(dst_hbm.at[wid], dstb)

    def gather(j, rows, sem):
        pltpu.async_copy(hp_hbm.at[srcb.at[pl.ds(j * K, K)]], rows, sem)

    def wait_gather(rows, sem):
        pltpu.make_async_copy(hp_hbm.at[srcb.at[pl.ds(0, K)]], rows, sem).wait()

    def scatter(j, rows, sem):
        pltpu.async_copy(rows, acc.at[dstb.at[j]], sem, add=True)

    def wait_scatter(rows, sem):
        pltpu.make_async_copy(rows, acc.at[dstb.at[0]], sem).wait()

    npairs = (nchunk - 1) // 2
    gather(0, rows0, g0)

    def pair(t, _):
        j = 2 * t
        wait_gather(rows0, g0)
        gather(j + 1, rows1, g1)
        scatter(j, rows0, s0)
        wait_gather(rows1, g1)
        wait_scatter(rows0, s0)
        gather(j + 2, rows0, g0)
        scatter(j + 1, rows1, s1)

        @pl.when(t + 1 < npairs)
        def _():
            wait_scatter(rows1, s1)

        return 0

    lax.fori_loop(0, npairs, pair, 0)
    wait_gather(rows0, g0)
    scatter(nchunk - 1, rows0, s0)
    wait_scatter(rows1, s1)
    wait_scatter(rows0, s0)
    plsc.subcore_barrier()

    @pl.when(cid == 0)
    def _():
        pltpu.sync_copy(acc.at[sl], out0.at[sl])

    @pl.when(cid == 1)
    def _():
        pltpu.sync_copy(acc.at[sl], out1.at[sl])


def _make_mp_kernel(nchunk):
    return functools.partial(
        pl.kernel,
        out_type=(
            jax.ShapeDtypeStruct((ACC_ROWS, D), jnp.float32),
            jax.ShapeDtypeStruct((ACC_ROWS, D), jnp.float32),
        ),
        mesh=_sc_mesh(),
        scratch_types=[
            pltpu.VMEM((nchunk * K,), jnp.int32),
            pltpu.VMEM((nchunk, K), jnp.int32),
            pltpu.VMEM((K, D), jnp.float32),
            pltpu.VMEM((K, D), jnp.float32),
            pltpu.VMEM_SHARED((ACC_ROWS, D), jnp.float32),
            pltpu.SemaphoreType.DMA,
            pltpu.SemaphoreType.DMA,
            pltpu.SemaphoreType.DMA,
            pltpu.SemaphoreType.DMA,
        ],
    )(functools.partial(_mp_body, nchunk))


def _dense1_body(x_ref, c0_ref, c1_ref, w_ref, hp_ref, dinv_ref):
    cnt = c0_ref[:, 0:1] + c1_ref[:, 0:1]
    dinv = lax.rsqrt(cnt + 2.0)
    h = jnp.dot(x_ref[...], w_ref[...], preferred_element_type=jnp.float32)
    hp_ref[...] = dinv * h
    dinv_ref[...] = dinv


def _post1_body(s0_ref, s1_ref, dinv_ref, b_ref, g_ref, be_ref, w_ref,
                x1_ref, hp2_ref):
    dinv = dinv_ref[...]
    t = dinv * (s0_ref[...] + s1_ref[...]) + b_ref[...]
    mu = jnp.mean(t, axis=1, keepdims=True)
    var = jnp.mean((t - mu) ** 2, axis=1, keepdims=True)
    tn = (t - mu) * lax.rsqrt(var + 1e-5) * g_ref[...] + be_ref[...]
    x1 = jnp.maximum(tn, 0.0)
    x1_ref[...] = x1
    h2 = jnp.dot(x1, w_ref[...], preferred_element_type=jnp.float32)
    hp2_ref[...] = dinv * h2


def _post2_body(s0_ref, s1_ref, dinv_ref, b_ref, g_ref, be_ref, x1_ref,
                wc_ref, bc_ref, out_ref):
    t = dinv_ref[...] * (s0_ref[...] + s1_ref[...]) + b_ref[...]
    mu = jnp.mean(t, axis=1, keepdims=True)
    var = jnp.mean((t - mu) ** 2, axis=1, keepdims=True)
    tn = (t - mu) * lax.rsqrt(var + 1e-5) * g_ref[...] + be_ref[...]
    h = jnp.maximum(tn, 0.0) + 0.2 * x1_ref[...]
    logits = jnp.dot(h, wc_ref[...], preferred_element_type=jnp.float32)
    logits = logits + bc_ref[...]
    m = jnp.max(logits, axis=1, keepdims=True)
    lse = jnp.log(jnp.sum(jnp.exp(logits - m), axis=1, keepdims=True)) + m
    out_ref[...] = logits - lse


def _row_spec(bs):
    return pl.BlockSpec(bs, lambda i: (i, 0))


def _full_spec(bs):
    return pl.BlockSpec(bs, lambda i: (0, 0))


def kernel(x, edge_index, W1, b1, g1, be1, W2, b2, g2, be2, Wc, bc):
    E = edge_index.shape[1]
    epad = ((E + NW * K - 1) // (NW * K)) * (NW * K)
    pad_n = epad - E
    nchunk = epad // (NW * K)
    if nchunk % 2 == 0:
        epad += NW * K
        pad_n += NW * K
        nchunk += 1

    ar = jnp.arange(pad_n, dtype=jnp.int32)
    src_pad = ar % jnp.int32(N)
    dst_pad = jnp.int32(N) + (ar % jnp.int32(PAD_ROWS))
    src = jnp.concatenate([edge_index[0], src_pad]).reshape(NW, nchunk * K)
    dst = jnp.concatenate([edge_index[1], dst_pad]).reshape(NW, nchunk, K)

    c0, c1 = _make_deg_kernel(nchunk)(dst)

    grid = N // BB
    W1t = W1.T
    W2t = W2.T
    Wct = Wc.T
    b1r, g1r, be1r = b1.reshape(1, H), g1.reshape(1, H), be1.reshape(1, H)
    b2r, g2r, be2r = b2.reshape(1, H), g2.reshape(1, H), be2.reshape(1, H)
    bcr = bc.reshape(1, O)

    hp1, dinv = pl.pallas_call(
        _dense1_body,
        grid=(grid,),
        in_specs=[_row_spec((BB, D)), _row_spec((BB, 16)), _row_spec((BB, 16)),
                  _full_spec((D, H))],
        out_specs=[_row_spec((BB, H)), _row_spec((BB, 1))],
        out_shape=[jax.ShapeDtypeStruct((ACC_ROWS, H), jnp.float32),
                   jax.ShapeDtypeStruct((N, 1), jnp.float32)],
    )(x, c0, c1, W1t)

    s0, s1 = _make_mp_kernel(nchunk)(hp1, src, dst)

    x1, hp2 = pl.pallas_call(
        _post1_body,
        grid=(grid,),
        in_specs=[_row_spec((BB, H)), _row_spec((BB, H)), _row_spec((BB, 1)),
                  _full_spec((1, H)), _full_spec((1, H)), _full_spec((1, H)),
                  _full_spec((H, H))],
        out_specs=[_row_spec((BB, H)), _row_spec((BB, H))],
        out_shape=[jax.ShapeDtypeStruct((N, H), jnp.float32),
                   jax.ShapeDtypeStruct((ACC_ROWS, H), jnp.float32)],
    )(s0, s1, dinv, b1r, g1r, be1r, W2t)

    t0, t1 = _make_mp_kernel(nchunk)(hp2, src, dst)

    out = pl.pallas_call(
        _post2_body,
        grid=(grid,),
        in_specs=[_row_spec((BB, H)), _row_spec((BB, H)), _row_spec((BB, 1)),
                  _full_spec((1, H)), _full_spec((1, H)), _full_spec((1, H)),
                  _row_spec((BB, H)), _full_spec((H, O)), _full_spec((1, O))],
        out_specs=_row_spec((BB, O)),
        out_shape=jax.ShapeDtypeStruct((N, O), jnp.float32),
    )(t0, t1, dinv, b2r, g2r, be2r, x1, Wct, bcr)

    return out


# speedup vs baseline: 32.0004x; 1.2557x over previous
import functools

import jax
import jax.numpy as jnp
from jax import lax
from jax.experimental import pallas as pl
from jax.experimental.pallas import tpu as pltpu
from jax.experimental.pallas import tpu_sc as plsc

N = 10000
D = 128
H = 128
O = 40
NC = 2
NS = 16
NW = NC * NS
K = 96
KS = 48
ROWS_PER_TILE = 632
ACC_ROWS = NS * ROWS_PER_TILE
PAD_ROWS = ACC_ROWS - N
BB = 1000


def _sc_mesh():
    return plsc.VectorSubcoreMesh(core_axis_name="c", subcore_axis_name="s")


def _deg_body(nchunk, dst_hbm, out0, out1, dstb, onesb, acc, zbuf, dsem):
    cid = lax.axis_index("c")
    sid = lax.axis_index("s")
    wid = sid * NC + cid

    def fill_ones(j, _):
        onesb[j] = jnp.full((16,), 1.0, jnp.float32)
        return 0

    lax.fori_loop(0, K, fill_ones, 0)

    def fill_zeros(j, _):
        zbuf[j] = jnp.zeros((16,), jnp.float32)
        return 0

    lax.fori_loop(0, ROWS_PER_TILE, fill_zeros, 0)
    sl = pl.ds(sid * ROWS_PER_TILE, ROWS_PER_TILE)
    pltpu.sync_copy(zbuf, acc.at[sl])
    plsc.subcore_barrier()

    pltpu.sync_copy(dst_hbm.at[wid], dstb)

    def body(j, _):
        pltpu.async_copy(onesb, acc.at[dstb.at[j]], dsem, add=True)

        @pl.when(j > 0)
        def _():
            pltpu.make_async_copy(onesb, acc.at[dstb.at[0]], dsem).wait()

        return 0

    lax.fori_loop(0, nchunk, body, 0)
    pltpu.make_async_copy(onesb, acc.at[dstb.at[0]], dsem).wait()
    plsc.subcore_barrier()

    @pl.when(cid == 0)
    def _():
        pltpu.sync_copy(acc.at[sl], out0.at[sl])

    @pl.when(cid == 1)
    def _():
        pltpu.sync_copy(acc.at[sl], out1.at[sl])


def _make_deg_kernel(nchunk):
    return functools.partial(
        pl.kernel,
        out_type=(
            jax.ShapeDtypeStruct((ACC_ROWS, 16), jnp.float32),
            jax.ShapeDtypeStruct((ACC_ROWS, 16), jnp.float32),
        ),
        mesh=_sc_mesh(),
        scratch_types=[
            pltpu.VMEM((nchunk, K), jnp.int32),
            pltpu.VMEM((K, 16), jnp.float32),
            pltpu.VMEM_SHARED((ACC_ROWS, 16), jnp.float32),
            pltpu.VMEM((ROWS_PER_TILE, 16), jnp.float32),
            pltpu.SemaphoreType.DMA,
        ],
    )(functools.partial(_deg_body, nchunk))


def _mp_body(nm, hp_hbm, src_hbm, dst_hbm, out0, out1,
             srcb, dstb, r0, r1, r2, r3, acc,
             g0, g1, g2, g3, s0, s1, s2, s3):
    cid = lax.axis_index("c")
    sid = lax.axis_index("s")
    wid = sid * NC + cid
    sl = pl.ds(sid * ROWS_PER_TILE, ROWS_PER_TILE)

    pltpu.sync_copy(hp_hbm.at[sl], acc.at[sl])
    plsc.subcore_barrier()

    pltpu.sync_copy(src_hbm.at[wid], srcb)
    pltpu.sync_copy(dst_hbm.at[wid], dstb)

    rows = (r0, r1, r2, r3)
    gsem = (g0, g1, g2, g3)
    ssem = (s0, s1, s2, s3)

    def gather(j, b):
        pltpu.async_copy(hp_hbm.at[srcb.at[pl.ds(j * KS, KS)]],
                         rows[b], gsem[b])

    def wait_gather(b):
        pltpu.make_async_copy(hp_hbm.at[srcb.at[pl.ds(0, KS)]],
                              rows[b], gsem[b]).wait()

    def scatter(j, b):
        pltpu.async_copy(rows[b], acc.at[dstb.at[pl.ds(j * KS, KS)]],
                         ssem[b], add=True)

    def wait_scatter(b):
        pltpu.make_async_copy(rows[b], acc.at[dstb.at[pl.ds(0, KS)]],
                              ssem[b]).wait()

    def step(j, b):
        wait_gather(b)
        scatter(j, b)
        bn = (b + 3) % 4

        @pl.when(j + 3 < nm)
        def _():
            @pl.when(j >= 1)
            def _():
                wait_scatter(bn)

            gather(j + 3, bn)

    gather(0, 0)
    gather(1, 1)
    gather(2, 2)

    quads = nm // 4

    def quad(t, _):
        j = 4 * t
        step(j, 0)
        step(j + 1, 1)
        step(j + 2, 2)
        step(j + 3, 3)
        return 0

    lax.fori_loop(0, quads, quad, 0)
    for j in range(4 * quads, nm):
        step(j, j % 4)
    for b in range(4):
        wait_scatter(b)
    plsc.subcore_barrier()

    @pl.when(cid == 0)
    def _():
        pltpu.sync_copy(acc.at[sl], out0.at[sl])

    @pl.when(cid == 1)
    def _():
        pltpu.sync_copy(acc.at[sl], out1.at[sl])


def _make_mp_kernel(nm):
    return functools.partial(
        pl.kernel,
        out_type=(
            jax.ShapeDtypeStruct((ACC_ROWS, D), jnp.float32),
            jax.ShapeDtypeStruct((ACC_ROWS, D), jnp.float32),
        ),
        mesh=_sc_mesh(),
        scratch_types=(
            [pltpu.VMEM((nm * KS,), jnp.int32),
             pltpu.VMEM((nm * KS,), jnp.int32)]
            + [pltpu.VMEM((KS, D), jnp.float32) for _ in range(4)]
            + [pltpu.VMEM_SHARED((ACC_ROWS, D), jnp.float32)]
            + [pltpu.SemaphoreType.DMA for _ in range(8)]
        ),
    )(functools.partial(_mp_body, nm))


def _dense1_body(x_ref, c0_ref, c1_ref, w_ref, hp_ref, dinv_ref):
    cnt = c0_ref[:, 0:1] + c1_ref[:, 0:1]
    dinv = lax.rsqrt(cnt + 2.0)
    h = jnp.dot(x_ref[...], w_ref[...], preferred_element_type=jnp.float32)
    hp_ref[...] = dinv * h
    dinv_ref[...] = dinv


def _post1_body(s0_ref, s1_ref, dinv_ref, b_ref, g_ref, be_ref, w_ref,
                x1_ref, hp2_ref):
    dinv = dinv_ref[...]
    t = dinv * (s0_ref[...] + s1_ref[...]) + b_ref[...]
    mu = jnp.mean(t, axis=1, keepdims=True)
    var = jnp.mean((t - mu) ** 2, axis=1, keepdims=True)
    tn = (t - mu) * lax.rsqrt(var + 1e-5) * g_ref[...] + be_ref[...]
    x1 = jnp.maximum(tn, 0.0)
    x1_ref[...] = x1
    h2 = jnp.dot(x1, w_ref[...], preferred_element_type=jnp.float32)
    hp2_ref[...] = dinv * h2


def _post2_body(s0_ref, s1_ref, dinv_ref, b_ref, g_ref, be_ref, x1_ref,
                wc_ref, bc_ref, out_ref):
    t = dinv_ref[...] * (s0_ref[...] + s1_ref[...]) + b_ref[...]
    mu = jnp.mean(t, axis=1, keepdims=True)
    var = jnp.mean((t - mu) ** 2, axis=1, keepdims=True)
    tn = (t - mu) * lax.rsqrt(var + 1e-5) * g_ref[...] + be_ref[...]
    h = jnp.maximum(tn, 0.0) + 0.2 * x1_ref[...]
    logits = jnp.dot(h, wc_ref[...], preferred_element_type=jnp.float32)
    logits = logits + bc_ref[...]
    m = jnp.max(logits, axis=1, keepdims=True)
    lse = jnp.log(jnp.sum(jnp.exp(logits - m), axis=1, keepdims=True)) + m
    out_ref[...] = logits - lse


def _row_spec(bs):
    return pl.BlockSpec(bs, lambda i: (i, 0))


def _full_spec(bs):
    return pl.BlockSpec(bs, lambda i: (0, 0))


def kernel(x, edge_index, W1, b1, g1, be1, W2, b2, g2, be2, Wc, bc):
    E = edge_index.shape[1]
    epad = ((E + NW * K - 1) // (NW * K)) * (NW * K)
    pad_n = epad - E
    nchunk = epad // (NW * K)
    nm = epad // (NW * KS)

    ar = jnp.arange(pad_n, dtype=jnp.int32)
    src_pad = ar % jnp.int32(N)
    dst_pad = jnp.int32(N) + (ar % jnp.int32(PAD_ROWS))
    srcf = jnp.concatenate([edge_index[0], src_pad]).reshape(NW, nm * KS)
    dstf = jnp.concatenate([edge_index[1], dst_pad]).reshape(NW, nm * KS)
    dst3 = dstf.reshape(NW, nchunk, K)

    c0, c1 = _make_deg_kernel(nchunk)(dst3)

    grid = N // BB
    W1t = W1.T
    W2t = W2.T
    Wct = Wc.T
    b1r, g1r, be1r = b1.reshape(1, H), g1.reshape(1, H), be1.reshape(1, H)
    b2r, g2r, be2r = b2.reshape(1, H), g2.reshape(1, H), be2.reshape(1, H)
    bcr = bc.reshape(1, O)

    hp1, dinv = pl.pallas_call(
        _dense1_body,
        grid=(grid,),
        in_specs=[_row_spec((BB, D)), _row_spec((BB, 16)), _row_spec((BB, 16)),
                  _full_spec((D, H))],
        out_specs=[_row_spec((BB, H)), _row_spec((BB, 1))],
        out_shape=[jax.ShapeDtypeStruct((ACC_ROWS, H), jnp.float32),
                   jax.ShapeDtypeStruct((N, 1), jnp.float32)],
    )(x, c0, c1, W1t)

    s0, s1 = _make_mp_kernel(nm)(hp1, srcf, dstf)

    x1, hp2 = pl.pallas_call(
        _post1_body,
        grid=(grid,),
        in_specs=[_row_spec((BB, H)), _row_spec((BB, H)), _row_spec((BB, 1)),
                  _full_spec((1, H)), _full_spec((1, H)), _full_spec((1, H)),
                  _full_spec((H, H))],
        out_specs=[_row_spec((BB, H)), _row_spec((BB, H))],
        out_shape=[jax.ShapeDtypeStruct((N, H), jnp.float32),
                   jax.ShapeDtypeStruct((ACC_ROWS, H), jnp.float32)],
    )(s0, s1, dinv, b1r, g1r, be1r, W2t)

    t0, t1 = _make_mp_kernel(nm)(hp2, srcf, dstf)

    out = pl.pallas_call(
        _post2_body,
        grid=(grid,),
        in_specs=[_row_spec((BB, H)), _row_spec((BB, H)), _row_spec((BB, 1)),
                  _full_spec((1, H)), _full_spec((1, H)), _full_spec((1, H)),
                  _row_spec((BB, H)), _full_spec((H, O)), _full_spec((1, O))],
        out_specs=_row_spec((BB, O)),
        out_shape=jax.ShapeDtypeStruct((N, O), jnp.float32),
    )(t0, t1, dinv, b2r, g2r, be2r, x1, Wct, bcr)

    return out


# speedup vs baseline: 33.6415x; 1.0513x over previous
import functools

import jax
import jax.numpy as jnp
from jax import lax
from jax.experimental import pallas as pl
from jax.experimental.pallas import tpu as pltpu
from jax.experimental.pallas import tpu_sc as plsc

N = 10000
D = 128
H = 128
O = 40
NC = 2
NS = 16
NW = NC * NS
K = 96
KS = 32
NB = 6
ROWS_PER_TILE = 632
ACC_ROWS = NS * ROWS_PER_TILE
PAD_ROWS = ACC_ROWS - N
BB = 1000


def _sc_mesh():
    return plsc.VectorSubcoreMesh(core_axis_name="c", subcore_axis_name="s")


def _deg_body(nchunk, dst_hbm, out0, out1, dstb, onesb, acc, zbuf, dsem):
    cid = lax.axis_index("c")
    sid = lax.axis_index("s")
    wid = sid * NC + cid

    def fill_ones(j, _):
        onesb[j] = jnp.full((16,), 1.0, jnp.float32)
        return 0

    lax.fori_loop(0, K, fill_ones, 0)

    def fill_zeros(j, _):
        zbuf[j] = jnp.zeros((16,), jnp.float32)
        return 0

    lax.fori_loop(0, ROWS_PER_TILE, fill_zeros, 0)
    sl = pl.ds(sid * ROWS_PER_TILE, ROWS_PER_TILE)
    pltpu.sync_copy(zbuf, acc.at[sl])
    plsc.subcore_barrier()

    pltpu.sync_copy(dst_hbm.at[wid], dstb)

    def body(j, _):
        pltpu.async_copy(onesb, acc.at[dstb.at[j]], dsem, add=True)

        @pl.when(j > 0)
        def _():
            pltpu.make_async_copy(onesb, acc.at[dstb.at[0]], dsem).wait()

        return 0

    lax.fori_loop(0, nchunk, body, 0)
    pltpu.make_async_copy(onesb, acc.at[dstb.at[0]], dsem).wait()
    plsc.subcore_barrier()

    @pl.when(cid == 0)
    def _():
        pltpu.sync_copy(acc.at[sl], out0.at[sl])

    @pl.when(cid == 1)
    def _():
        pltpu.sync_copy(acc.at[sl], out1.at[sl])


def _make_deg_kernel(nchunk):
    return functools.partial(
        pl.kernel,
        out_type=(
            jax.ShapeDtypeStruct((ACC_ROWS, 16), jnp.float32),
            jax.ShapeDtypeStruct((ACC_ROWS, 16), jnp.float32),
        ),
        mesh=_sc_mesh(),
        scratch_types=[
            pltpu.VMEM((nchunk, K), jnp.int32),
            pltpu.VMEM((K, 16), jnp.float32),
            pltpu.VMEM_SHARED((ACC_ROWS, 16), jnp.float32),
            pltpu.VMEM((ROWS_PER_TILE, 16), jnp.float32),
            pltpu.SemaphoreType.DMA,
        ],
    )(functools.partial(_deg_body, nchunk))


def _mp_body(nm, hp_hbm, src_hbm, dst_hbm, out0, out1, *scr):
    srcb, dstb = scr[0], scr[1]
    rows = scr[2:2 + NB]
    acc = scr[2 + NB]
    gsem = scr[3 + NB:3 + 2 * NB]
    ssem = scr[3 + 2 * NB:3 + 3 * NB]
    cid = lax.axis_index("c")
    sid = lax.axis_index("s")
    wid = sid * NC + cid
    sl = pl.ds(sid * ROWS_PER_TILE, ROWS_PER_TILE)

    pltpu.sync_copy(hp_hbm.at[sl], acc.at[sl])
    plsc.subcore_barrier()

    pltpu.sync_copy(src_hbm.at[wid], srcb)
    pltpu.sync_copy(dst_hbm.at[wid], dstb)

    def gather(j, b):
        pltpu.async_copy(hp_hbm.at[srcb.at[pl.ds(j * KS, KS)]],
                         rows[b], gsem[b])

    def wait_gather(b):
        pltpu.make_async_copy(hp_hbm.at[srcb.at[pl.ds(0, KS)]],
                              rows[b], gsem[b]).wait()

    def scatter(j, b):
        pltpu.async_copy(rows[b], acc.at[dstb.at[pl.ds(j * KS, KS)]],
                         ssem[b], add=True)

    def wait_scatter(b):
        pltpu.make_async_copy(rows[b], acc.at[dstb.at[pl.ds(0, KS)]],
                              ssem[b]).wait()

    def step(j, b):
        wait_gather(b)
        scatter(j, b)
        bn = (b + NB - 1) % NB

        @pl.when(j + NB - 1 < nm)
        def _():
            @pl.when(j >= 1)
            def _():
                wait_scatter(bn)

            gather(j + NB - 1, bn)

    for c in range(NB - 1):
        gather(c, c)

    groups = nm // NB

    def group(t, _):
        j = NB * t
        for i in range(NB):
            step(j + i, i)
        return 0

    lax.fori_loop(0, groups, group, 0)
    for j in range(NB * groups, nm):
        step(j, j % NB)
    for b in range(NB):
        wait_scatter(b)
    plsc.subcore_barrier()

    @pl.when(cid == 0)
    def _():
        pltpu.sync_copy(acc.at[sl], out0.at[sl])

    @pl.when(cid == 1)
    def _():
        pltpu.sync_copy(acc.at[sl], out1.at[sl])


def _make_mp_kernel(nm):
    return functools.partial(
        pl.kernel,
        out_type=(
            jax.ShapeDtypeStruct((ACC_ROWS, D), jnp.float32),
            jax.ShapeDtypeStruct((ACC_ROWS, D), jnp.float32),
        ),
        mesh=_sc_mesh(),
        scratch_types=(
            [pltpu.VMEM((nm * KS,), jnp.int32),
             pltpu.VMEM((nm * KS,), jnp.int32)]
            + [pltpu.VMEM((KS, D), jnp.float32) for _ in range(NB)]
            + [pltpu.VMEM_SHARED((ACC_ROWS, D), jnp.float32)]
            + [pltpu.SemaphoreType.DMA for _ in range(2 * NB)]
        ),
    )(functools.partial(_mp_body, nm))


def _dense1_body(x_ref, c0_ref, c1_ref, w_ref, hp_ref, dinv_ref):
    cnt = c0_ref[:, 0:1] + c1_ref[:, 0:1]
    dinv = lax.rsqrt(cnt + 2.0)
    h = jnp.dot(x_ref[...], w_ref[...], preferred_element_type=jnp.float32)
    hp_ref[...] = dinv * h
    dinv_ref[...] = dinv


def _post1_body(s0_ref, s1_ref, dinv_ref, b_ref, g_ref, be_ref, w_ref,
                x1_ref, hp2_ref):
    dinv = dinv_ref[...]
    t = dinv * (s0_ref[...] + s1_ref[...]) + b_ref[...]
    mu = jnp.mean(t, axis=1, keepdims=True)
    var = jnp.mean((t - mu) ** 2, axis=1, keepdims=True)
    tn = (t - mu) * lax.rsqrt(var + 1e-5) * g_ref[...] + be_ref[...]
    x1 = jnp.maximum(tn, 0.0)
    x1_ref[...] = x1
    h2 = jnp.dot(x1, w_ref[...], preferred_element_type=jnp.float32)
    hp2_ref[...] = dinv * h2


def _post2_body(s0_ref, s1_ref, dinv_ref, b_ref, g_ref, be_ref, x1_ref,
                wc_ref, bc_ref, out_ref):
    t = dinv_ref[...] * (s0_ref[...] + s1_ref[...]) + b_ref[...]
    mu = jnp.mean(t, axis=1, keepdims=True)
    var = jnp.mean((t - mu) ** 2, axis=1, keepdims=True)
    tn = (t - mu) * lax.rsqrt(var + 1e-5) * g_ref[...] + be_ref[...]
    h = jnp.maximum(tn, 0.0) + 0.2 * x1_ref[...]
    logits = jnp.dot(h, wc_ref[...], preferred_element_type=jnp.float32)
    logits = logits + bc_ref[...]
    m = jnp.max(logits, axis=1, keepdims=True)
    lse = jnp.log(jnp.sum(jnp.exp(logits - m), axis=1, keepdims=True)) + m
    out_ref[...] = logits - lse


def _row_spec(bs):
    return pl.BlockSpec(bs, lambda i: (i, 0))


def _full_spec(bs):
    return pl.BlockSpec(bs, lambda i: (0, 0))


def kernel(x, edge_index, W1, b1, g1, be1, W2, b2, g2, be2, Wc, bc):
    E = edge_index.shape[1]
    epad = ((E + NW * K - 1) // (NW * K)) * (NW * K)
    pad_n = epad - E
    nchunk = epad // (NW * K)
    nm = epad // (NW * KS)

    ar = jnp.arange(pad_n, dtype=jnp.int32)
    src_pad = ar % jnp.int32(N)
    dst_pad = jnp.int32(N) + (ar % jnp.int32(PAD_ROWS))
    srcf = jnp.concatenate([edge_index[0], src_pad]).reshape(NW, nm * KS)
    dstf = jnp.concatenate([edge_index[1], dst_pad]).reshape(NW, nm * KS)
    dst3 = dstf.reshape(NW, nchunk, K)

    c0, c1 = _make_deg_kernel(nchunk)(dst3)

    grid = N // BB
    W1t = W1.T
    W2t = W2.T
    Wct = Wc.T
    b1r, g1r, be1r = b1.reshape(1, H), g1.reshape(1, H), be1.reshape(1, H)
    b2r, g2r, be2r = b2.reshape(1, H), g2.reshape(1, H), be2.reshape(1, H)
    bcr = bc.reshape(1, O)

    hp1, dinv = pl.pallas_call(
        _dense1_body,
        grid=(grid,),
        in_specs=[_row_spec((BB, D)), _row_spec((BB, 16)), _row_spec((BB, 16)),
                  _full_spec((D, H))],
        out_specs=[_row_spec((BB, H)), _row_spec((BB, 1))],
        out_shape=[jax.ShapeDtypeStruct((ACC_ROWS, H), jnp.float32),
                   jax.ShapeDtypeStruct((N, 1), jnp.float32)],
    )(x, c0, c1, W1t)

    s0, s1 = _make_mp_kernel(nm)(hp1, srcf, dstf)

    x1, hp2 = pl.pallas_call(
        _post1_body,
        grid=(grid,),
        in_specs=[_row_spec((BB, H)), _row_spec((BB, H)), _row_spec((BB, 1)),
                  _full_spec((1, H)), _full_spec((1, H)), _full_spec((1, H)),
                  _full_spec((H, H))],
        out_specs=[_row_spec((BB, H)), _row_spec((BB, H))],
        out_shape=[jax.ShapeDtypeStruct((N, H), jnp.float32),
                   jax.ShapeDtypeStruct((ACC_ROWS, H), jnp.float32)],
    )(s0, s1, dinv, b1r, g1r, be1r, W2t)

    t0, t1 = _make_mp_kernel(nm)(hp2, srcf, dstf)

    out = pl.pallas_call(
        _post2_body,
        grid=(grid,),
        in_specs=[_row_spec((BB, H)), _row_spec((BB, H)), _row_spec((BB, 1)),
                  _full_spec((1, H)), _full_spec((1, H)), _full_spec((1, H)),
                  _row_spec((BB, H)), _full_spec((H, O)), _full_spec((1, O))],
        out_specs=_row_spec((BB, O)),
        out_shape=jax.ShapeDtypeStruct((N, O), jnp.float32),
    )(t0, t1, dinv, b2r, g2r, be2r, x1, Wct, bcr)

    return out


# speedup vs baseline: 34.4394x; 1.0237x over previous
import functools

import jax
import jax.numpy as jnp
from jax import lax
from jax.experimental import pallas as pl
from jax.experimental.pallas import tpu as pltpu
from jax.experimental.pallas import tpu_sc as plsc

N = 10000
D = 128
H = 128
O = 40
NC = 2
NS = 16
NW = NC * NS
K = 96
KS = 24
NB = 8
ROWS_PER_TILE = 632
ACC_ROWS = NS * ROWS_PER_TILE
PAD_ROWS = ACC_ROWS - N
BB = 1000


def _sc_mesh():
    return plsc.VectorSubcoreMesh(core_axis_name="c", subcore_axis_name="s")


def _deg_body(nchunk, dst_hbm, out0, out1, dstb, onesb, acc, zbuf, dsem):
    cid = lax.axis_index("c")
    sid = lax.axis_index("s")
    wid = sid * NC + cid

    def fill_ones(j, _):
        onesb[j] = jnp.full((16,), 1.0, jnp.float32)
        return 0

    lax.fori_loop(0, K, fill_ones, 0)

    def fill_zeros(j, _):
        zbuf[j] = jnp.zeros((16,), jnp.float32)
        return 0

    lax.fori_loop(0, ROWS_PER_TILE, fill_zeros, 0)
    sl = pl.ds(sid * ROWS_PER_TILE, ROWS_PER_TILE)
    pltpu.sync_copy(zbuf, acc.at[sl])
    plsc.subcore_barrier()

    pltpu.sync_copy(dst_hbm.at[wid], dstb)

    def body(j, _):
        pltpu.async_copy(onesb, acc.at[dstb.at[j]], dsem, add=True)

        @pl.when(j > 0)
        def _():
            pltpu.make_async_copy(onesb, acc.at[dstb.at[0]], dsem).wait()

        return 0

    lax.fori_loop(0, nchunk, body, 0)
    pltpu.make_async_copy(onesb, acc.at[dstb.at[0]], dsem).wait()
    plsc.subcore_barrier()

    @pl.when(cid == 0)
    def _():
        pltpu.sync_copy(acc.at[sl], out0.at[sl])

    @pl.when(cid == 1)
    def _():
        pltpu.sync_copy(acc.at[sl], out1.at[sl])


def _make_deg_kernel(nchunk):
    return functools.partial(
        pl.kernel,
        out_type=(
            jax.ShapeDtypeStruct((ACC_ROWS, 16), jnp.float32),
            jax.ShapeDtypeStruct((ACC_ROWS, 16), jnp.float32),
        ),
        mesh=_sc_mesh(),
        scratch_types=[
            pltpu.VMEM((nchunk, K), jnp.int32),
            pltpu.VMEM((K, 16), jnp.float32),
            pltpu.VMEM_SHARED((ACC_ROWS, 16), jnp.float32),
            pltpu.VMEM((ROWS_PER_TILE, 16), jnp.float32),
            pltpu.SemaphoreType.DMA,
        ],
    )(functools.partial(_deg_body, nchunk))


def _mp_body(nm, hp_hbm, src_hbm, dst_hbm, out0, out1, *scr):
    srcb, dstb = scr[0], scr[1]
    rows = scr[2:2 + NB]
    acc = scr[2 + NB]
    gsem = scr[3 + NB:3 + 2 * NB]
    ssem = scr[3 + 2 * NB:3 + 3 * NB]
    cid = lax.axis_index("c")
    sid = lax.axis_index("s")
    wid = sid * NC + cid
    sl = pl.ds(sid * ROWS_PER_TILE, ROWS_PER_TILE)

    pltpu.sync_copy(hp_hbm.at[sl], acc.at[sl])
    plsc.subcore_barrier()

    pltpu.sync_copy(src_hbm.at[wid], srcb)
    pltpu.sync_copy(dst_hbm.at[wid], dstb)

    def gather(j, b):
        pltpu.async_copy(hp_hbm.at[srcb.at[pl.ds(j * KS, KS)]],
                         rows[b], gsem[b])

    def wait_gather(b):
        pltpu.make_async_copy(hp_hbm.at[srcb.at[pl.ds(0, KS)]],
                              rows[b], gsem[b]).wait()

    def scatter(j, b):
        pltpu.async_copy(rows[b], acc.at[dstb.at[pl.ds(j * KS, KS)]],
                         ssem[b], add=True)

    def wait_scatter(b):
        pltpu.make_async_copy(rows[b], acc.at[dstb.at[pl.ds(0, KS)]],
                              ssem[b]).wait()

    def step(j, b):
        wait_gather(b)
        scatter(j, b)
        bn = (b + NB - 1) % NB

        @pl.when(j + NB - 1 < nm)
        def _():
            @pl.when(j >= 1)
            def _():
                wait_scatter(bn)

            gather(j + NB - 1, bn)

    for c in range(NB - 1):
        gather(c, c)

    groups = nm // NB

    def group(t, _):
        j = NB * t
        for i in range(NB):
            step(j + i, i)
        return 0

    lax.fori_loop(0, groups, group, 0)
    for j in range(NB * groups, nm):
        step(j, j % NB)
    for b in range(NB):
        wait_scatter(b)
    plsc.subcore_barrier()

    @pl.when(cid == 0)
    def _():
        pltpu.sync_copy(acc.at[sl], out0.at[sl])

    @pl.when(cid == 1)
    def _():
        pltpu.sync_copy(acc.at[sl], out1.at[sl])


def _make_mp_kernel(nm):
    return functools.partial(
        pl.kernel,
        out_type=(
            jax.ShapeDtypeStruct((ACC_ROWS, D), jnp.float32),
            jax.ShapeDtypeStruct((ACC_ROWS, D), jnp.float32),
        ),
        mesh=_sc_mesh(),
        scratch_types=(
            [pltpu.VMEM((nm * KS,), jnp.int32),
             pltpu.VMEM((nm * KS,), jnp.int32)]
            + [pltpu.VMEM((KS, D), jnp.float32) for _ in range(NB)]
            + [pltpu.VMEM_SHARED((ACC_ROWS, D), jnp.float32)]
            + [pltpu.SemaphoreType.DMA for _ in range(2 * NB)]
        ),
    )(functools.partial(_mp_body, nm))


def _dense1_body(x_ref, c0_ref, c1_ref, w_ref, hp_ref, dinv_ref):
    cnt = c0_ref[:, 0:1] + c1_ref[:, 0:1]
    dinv = lax.rsqrt(cnt + 2.0)
    h = jnp.dot(x_ref[...], w_ref[...], preferred_element_type=jnp.float32)
    hp_ref[...] = dinv * h
    dinv_ref[...] = dinv


def _post1_body(s0_ref, s1_ref, dinv_ref, b_ref, g_ref, be_ref, w_ref,
                x1_ref, hp2_ref):
    dinv = dinv_ref[...]
    t = dinv * (s0_ref[...] + s1_ref[...]) + b_ref[...]
    mu = jnp.mean(t, axis=1, keepdims=True)
    var = jnp.mean((t - mu) ** 2, axis=1, keepdims=True)
    tn = (t - mu) * lax.rsqrt(var + 1e-5) * g_ref[...] + be_ref[...]
    x1 = jnp.maximum(tn, 0.0)
    x1_ref[...] = x1
    h2 = jnp.dot(x1, w_ref[...], preferred_element_type=jnp.float32)
    hp2_ref[...] = dinv * h2


def _post2_body(s0_ref, s1_ref, dinv_ref, b_ref, g_ref, be_ref, x1_ref,
                wc_ref, bc_ref, out_ref):
    t = dinv_ref[...] * (s0_ref[...] + s1_ref[...]) + b_ref[...]
    mu = jnp.mean(t, axis=1, keepdims=True)
    var = jnp.mean((t - mu) ** 2, axis=1, keepdims=True)
    tn = (t - mu) * lax.rsqrt(var + 1e-5) * g_ref[...] + be_ref[...]
    h = jnp.maximum(tn, 0.0) + 0.2 * x1_ref[...]
    logits = jnp.dot(h, wc_ref[...], preferred_element_type=jnp.float32)
    logits = logits + bc_ref[...]
    m = jnp.max(logits, axis=1, keepdims=True)
    lse = jnp.log(jnp.sum(jnp.exp(logits - m), axis=1, keepdims=True)) + m
    out_ref[...] = logits - lse


def _row_spec(bs):
    return pl.BlockSpec(bs, lambda i: (i, 0))


def _full_spec(bs):
    return pl.BlockSpec(bs, lambda i: (0, 0))


def kernel(x, edge_index, W1, b1, g1, be1, W2, b2, g2, be2, Wc, bc):
    E = edge_index.shape[1]
    epad = ((E + NW * K - 1) // (NW * K)) * (NW * K)
    pad_n = epad - E
    nchunk = epad // (NW * K)
    nm = epad // (NW * KS)

    ar = jnp.arange(pad_n, dtype=jnp.int32)
    src_pad = ar % jnp.int32(N)
    dst_pad = jnp.int32(N) + (ar % jnp.int32(PAD_ROWS))
    srcf = jnp.concatenate([edge_index[0], src_pad]).reshape(NW, nm * KS)
    dstf = jnp.concatenate([edge_index[1], dst_pad]).reshape(NW, nm * KS)
    dst3 = dstf.reshape(NW, nchunk, K)

    c0, c1 = _make_deg_kernel(nchunk)(dst3)

    grid = N // BB
    W1t = W1.T
    W2t = W2.T
    Wct = Wc.T
    b1r, g1r, be1r = b1.reshape(1, H), g1.reshape(1, H), be1.reshape(1, H)
    b2r, g2r, be2r = b2.reshape(1, H), g2.reshape(1, H), be2.reshape(1, H)
    bcr = bc.reshape(1, O)

    hp1, dinv = pl.pallas_call(
        _dense1_body,
        grid=(grid,),
        in_specs=[_row_spec((BB, D)), _row_spec((BB, 16)), _row_spec((BB, 16)),
                  _full_spec((D, H))],
        out_specs=[_row_spec((BB, H)), _row_spec((BB, 1))],
        out_shape=[jax.ShapeDtypeStruct((ACC_ROWS, H), jnp.float32),
                   jax.ShapeDtypeStruct((N, 1), jnp.float32)],
    )(x, c0, c1, W1t)

    s0, s1 = _make_mp_kernel(nm)(hp1, srcf, dstf)

    x1, hp2 = pl.pallas_call(
        _post1_body,
        grid=(grid,),
        in_specs=[_row_spec((BB, H)), _row_spec((BB, H)), _row_spec((BB, 1)),
                  _full_spec((1, H)), _full_spec((1, H)), _full_spec((1, H)),
                  _full_spec((H, H))],
        out_specs=[_row_spec((BB, H)), _row_spec((BB, H))],
        out_shape=[jax.ShapeDtypeStruct((N, H), jnp.float32),
                   jax.ShapeDtypeStruct((ACC_ROWS, H), jnp.float32)],
    )(s0, s1, dinv, b1r, g1r, be1r, W2t)

    t0, t1 = _make_mp_kernel(nm)(hp2, srcf, dstf)

    out = pl.pallas_call(
        _post2_body,
        grid=(grid,),
        in_specs=[_row_spec((BB, H)), _row_spec((BB, H)), _row_spec((BB, 1)),
                  _full_spec((1, H)), _full_spec((1, H)), _full_spec((1, H)),
                  _row_spec((BB, H)), _full_spec((H, O)), _full_spec((1, O))],
        out_specs=_row_spec((BB, O)),
        out_shape=jax.ShapeDtypeStruct((N, O), jnp.float32),
    )(t0, t1, dinv, b2r, g2r, be2r, x1, Wct, bcr)

    return out
